# baseline (speedup 1.0000x reference)
import functools

import jax
import jax.numpy as jnp
from jax import lax
from jax.experimental import pallas as pl
from jax.experimental.pallas import tpu as pltpu
from jax.experimental.pallas import tpu_sc as plsc

VOCAB = 1000000
EMBED_DIM = 32
BATCH = 4096
HIST_LEN = 50
TOTAL = BATCH * HIST_LEN
PACK = 128 // EMBED_DIM
CHUNK = 128
NC = 2
NS = 16
NW = NC * NS
ROWS_PER_W = TOTAL // NW
N_CHUNKS = ROWS_PER_W // CHUNK
L = 16


@functools.cache
def _make_kernel():
    mesh = plsc.VectorSubcoreMesh(core_axis_name="c", subcore_axis_name="s")

    @functools.partial(
        pl.kernel,
        mesh=mesh,
        out_type=jax.ShapeDtypeStruct((TOTAL // PACK, CHUNK), jnp.float32),
        scratch_types=[
            pltpu.VMEM((N_CHUNKS, CHUNK), jnp.int32),
            pltpu.VMEM((N_CHUNKS, CHUNK), jnp.int32),
            pltpu.VMEM((CHUNK, CHUNK), jnp.float32),
            pltpu.VMEM((CHUNK, CHUNK), jnp.float32),
            pltpu.VMEM((CHUNK // PACK, CHUNK), jnp.float32),
            pltpu.VMEM((CHUNK // PACK, CHUNK), jnp.float32),
            pltpu.SemaphoreType.DMA,
            pltpu.SemaphoreType.DMA,
            pltpu.SemaphoreType.DMA,
            pltpu.SemaphoreType.DMA,
        ],
        compiler_params=pltpu.CompilerParams(
            use_tc_tiling_on_sc=False, needs_layout_passes=False),
    )
    def emb_kernel(tbl_hbm, pidx_hbm, sub_hbm, out_hbm, pidx_v, sub_v,
                   gbuf0, gbuf1, obuf0, obuf1, gsem0, gsem1, ssem0, ssem1):
        gbufs = (gbuf0, gbuf1)
        obufs = (obuf0, obuf1)
        gsems = (gsem0, gsem1)
        ssems = (ssem0, ssem1)
        wid = lax.axis_index("s") * NC + lax.axis_index("c")
        pltpu.sync_copy(pidx_hbm.at[wid], pidx_v)
        pltpu.sync_copy(sub_hbm.at[wid], sub_v)
        obase = wid * (ROWS_PER_W // PACK)

        lanes = lax.iota(jnp.int32, L)
        drow_in_group = lanes // PACK
        dcol_base = (lanes % PACK) * EMBED_DIM

        def fire_gather(j, b):
            pltpu.async_copy(
                tbl_hbm.at[pidx_v.at[j]], gbufs[b], gsems[b])

        def drain_gather(b):
            pltpu.make_async_copy(
                tbl_hbm.at[pl.ds(0, CHUNK)], gbufs[b], gsems[b]).wait()

        def fire_store(j, b):
            pltpu.async_copy(
                obufs[b],
                out_hbm.at[pl.ds(obase + j * (CHUNK // PACK), CHUNK // PACK)],
                ssems[b])

        def wait_store(b):
            pltpu.make_async_copy(
                obufs[b], out_hbm.at[pl.ds(0, CHUNK // PACK)], ssems[b]).wait()

        def select(j, b):
            for g in range(CHUNK // L):
                srows = g * L + lanes
                scol = sub_v[j, pl.ds(g * L, L)]
                drows = g * (L // PACK) + drow_in_group
                for c in range(EMBED_DIM):
                    vals = plsc.load_gather(gbufs[b], [srows, scol + c])
                    plsc.store_scatter(obufs[b], [drows, dcol_base + c], vals)

        fire_gather(0, 0)
        drain_gather(0)
        fire_gather(1, 1)
        select(0, 0)
        fire_store(0, 0)
        drain_gather(1)
        fire_gather(2, 0)
        select(1, 1)
        fire_store(1, 1)

        @pl.loop(2, N_CHUNKS - 2, step=2)
        def _steady(o):
            for i in range(2):
                j = o + i
                b = i
                drain_gather(b)
                fire_gather(j + 1, 1 - b)
                wait_store(b)
                select(j, b)
                fire_store(j, b)

        drain_gather(0)
        fire_gather(N_CHUNKS - 1, 1)
        wait_store(0)
        select(N_CHUNKS - 2, 0)
        fire_store(N_CHUNKS - 2, 0)
        drain_gather(1)
        wait_store(1)
        select(N_CHUNKS - 1, 1)
        fire_store(N_CHUNKS - 1, 1)
        wait_store(0)
        wait_store(1)

    return emb_kernel


def kernel(input, emb_table):
    flat = input.astype(jnp.int32).reshape(NW, N_CHUNKS, CHUNK)
    pidx = flat // PACK
    sub = (flat % PACK) * EMBED_DIM
    tbl = emb_table.reshape(VOCAB // PACK, EMBED_DIM * PACK)
    out = _make_kernel()(tbl, pidx, sub)
    return out.reshape(BATCH, HIST_LEN, EMBED_DIM)


# speedup vs baseline: 1.2676x; 1.2676x over previous
import functools

import jax
import jax.numpy as jnp
from jax import lax
from jax.experimental import pallas as pl
from jax.experimental.pallas import tpu as pltpu
from jax.experimental.pallas import tpu_sc as plsc

VOCAB = 1000000
EMBED_DIM = 32
BATCH = 4096
HIST_LEN = 50
WIDE = 128
NC = 2
NS = 16
NW = NC * NS
BPW = BATCH // NW
L = 16


@functools.cache
def _make_kernel():
    mesh = plsc.VectorSubcoreMesh(core_axis_name="c", subcore_axis_name="s")

    @functools.partial(
        pl.kernel,
        mesh=mesh,
        out_type=jax.ShapeDtypeStruct((HIST_LEN, EMBED_DIM, BATCH), jnp.float32),
        scratch_types=[
            pltpu.VMEM((HIST_LEN, BPW), jnp.int32),
            pltpu.VMEM((BPW, WIDE), jnp.float32),
            pltpu.VMEM((BPW, WIDE), jnp.float32),
            pltpu.VMEM((EMBED_DIM, BPW), jnp.float32),
            pltpu.VMEM((EMBED_DIM, BPW), jnp.float32),
            pltpu.SemaphoreType.DMA,
            pltpu.SemaphoreType.DMA,
            pltpu.SemaphoreType.DMA,
            pltpu.SemaphoreType.DMA,
        ],
        compiler_params=pltpu.CompilerParams(
            use_tc_tiling_on_sc=False, needs_layout_passes=False),
    )
    def emb_kernel(tbl_hbm, idx_hbm, out_hbm, idx_v, gbuf0, gbuf1,
                   obuf0, obuf1, gsem0, gsem1, ssem0, ssem1):
        gbufs = (gbuf0, gbuf1)
        obufs = (obuf0, obuf1)
        gsems = (gsem0, gsem1)
        ssems = (ssem0, ssem1)
        wid = lax.axis_index("s") * NC + lax.axis_index("c")
        pltpu.sync_copy(idx_hbm.at[wid], idx_v)
        b0 = wid * BPW

        lanes = lax.iota(jnp.int32, L)

        def fire_gather(j, b):
            pltpu.async_copy(tbl_hbm.at[idx_v.at[j]], gbufs[b], gsems[b])

        def drain_gather(b):
            pltpu.make_async_copy(
                tbl_hbm.at[pl.ds(0, BPW)], gbufs[b], gsems[b]).wait()

        def fire_store(j, b):
            pltpu.async_copy(
                obufs[b], out_hbm.at[j, :, pl.ds(b0, BPW)], ssems[b])

        def wait_store(b):
            pltpu.make_async_copy(
                obufs[b], out_hbm.at[0, :, pl.ds(0, BPW)], ssems[b]).wait()

        def select(j, b):
            del j
            for g in range(BPW // L):
                srows = g * L + lanes
                for f in range(EMBED_DIM):
                    vals = plsc.load_gather(gbufs[b], [srows, lanes * 0 + f])
                    obufs[b][f, pl.ds(g * L, L)] = vals

        fire_gather(0, 0)
        drain_gather(0)
        fire_gather(1, 1)
        select(0, 0)
        fire_store(0, 0)
        drain_gather(1)
        fire_gather(2, 0)
        select(1, 1)
        fire_store(1, 1)

        @pl.loop(2, HIST_LEN - 2, step=2)
        def _steady(o):
            for i in range(2):
                j = o + i
                b = i
                drain_gather(b)
                fire_gather(j + 1, 1 - b)
                wait_store(b)
                select(j, b)
                fire_store(j, b)

        drain_gather(0)
        fire_gather(HIST_LEN - 1, 1)
        wait_store(0)
        select(HIST_LEN - 2, 0)
        fire_store(HIST_LEN - 2, 0)
        drain_gather(1)
        wait_store(1)
        select(HIST_LEN - 1, 1)
        fire_store(HIST_LEN - 1, 1)
        wait_store(0)
        wait_store(1)

    return emb_kernel


def kernel(input, emb_table):
    tblp = jnp.pad(emb_table, ((0, 0), (0, WIDE - EMBED_DIM)))
    idx = (input.astype(jnp.int32)
           .reshape(NW, BPW, HIST_LEN)
           .transpose(0, 2, 1))
    out = _make_kernel()(tblp, idx)
    return out.transpose(2, 0, 1)
